# baseline (speedup 1.0000x reference)
import functools

import jax
import jax.numpy as jnp
from jax import lax
from jax.experimental import pallas as pl
from jax.experimental.pallas import tpu as pltpu
from jax.experimental.pallas import tpu_sc as plsc

D = 128
L = 16
NC = 2
NS = 16
NW = NC * NS
K = 80


def _mm_body(x_ref, w_ref, o_ref):
    o_ref[...] = jnp.dot(x_ref[...], w_ref[...], preferred_element_type=jnp.float32)


def _matmul(x, w):
    n = x.shape[0]
    bm = 2000
    return pl.pallas_call(
        _mm_body,
        grid=(n // bm,),
        in_specs=[
            pl.BlockSpec((bm, D), lambda i: (i, 0)),
            pl.BlockSpec((D, D), lambda i: (0, 0)),
        ],
        out_specs=pl.BlockSpec((bm, D), lambda i: (i, 0)),
        out_shape=jax.ShapeDtypeStruct((n, D), jnp.float32),
    )(x, w)


def _combine_body(eps_ref, p_ref, wx_ref, o_ref):
    agg = p_ref[0].astype(jnp.float32) + p_ref[1].astype(jnp.float32)
    o_ref[...] = agg + eps_ref[0, 0] * wx_ref[...]


def _combine(p, wx, eps):
    n = wx.shape[0]
    bm = 1000
    return pl.pallas_call(
        _combine_body,
        grid=(n // bm,),
        in_specs=[
            pl.BlockSpec(memory_space=pltpu.SMEM),
            pl.BlockSpec((2, bm, D), lambda i: (0, i, 0)),
            pl.BlockSpec((bm, D), lambda i: (i, 0)),
        ],
        out_specs=pl.BlockSpec((bm, D), lambda i: (i, 0)),
        out_shape=jax.ShapeDtypeStruct((n, D), jnp.float32),
    )(eps, p, wx)


def _make_sc_scatter(npad, chunks):
    rows_per_tile = npad // NS
    mesh = plsc.VectorSubcoreMesh(core_axis_name="c", subcore_axis_name="s")

    @functools.partial(
        pl.kernel,
        out_type=jax.ShapeDtypeStruct((NC, npad, D), jnp.float32),
        mesh=mesh,
        scratch_types=[
            pltpu.VMEM((5, 2, K), jnp.int32),
            pltpu.VMEM((4, K, D), jnp.float32),
            pltpu.VMEM_SHARED((npad, D), jnp.float32),
            pltpu.SemaphoreType.DMA((5,)),
            pltpu.SemaphoreType.DMA((4,)),
            pltpu.SemaphoreType.DMA((4,)),
        ],
    )
    def sc_scatter(wx_hbm, rc_hbm, out_hbm, ibuf, gbuf, acc, isem, gsem, ssem):
        c = lax.axis_index("c")
        s = lax.axis_index("s")
        w = s * NC + c
        cbase = w * chunks

        def _idx_start(j, q):
            pltpu.async_copy(rc_hbm.at[cbase + j], ibuf.at[q], isem.at[q])

        def _idx_wait(j, q):
            pltpu.make_async_copy(rc_hbm.at[cbase + j], ibuf.at[q], isem.at[q]).wait()

        def _gather_start(j, q, p):
            pltpu.async_copy(wx_hbm.at[ibuf.at[q, 1]], gbuf.at[p], gsem.at[p])

        def _gather_wait(j, q, p):
            pltpu.make_async_copy(wx_hbm.at[ibuf.at[q, 1]], gbuf.at[p], gsem.at[p]).wait()

        def _scatter_desc(q, p):
            return pltpu.make_async_copy(gbuf.at[p], acc.at[ibuf.at[q, 0]], ssem.at[p])

        _idx_start(0, 0)
        _idx_start(1, 1)
        _idx_wait(0, 0)
        _gather_start(0, 0, 0)
        _idx_wait(1, 1)
        _gather_start(1, 1, 1)
        _idx_start(2, 2)

        def _zrow(i, carry):
            def _zcol(j, carry2):
                gbuf[2, i, pl.ds(j * L, L)] = jnp.zeros((L,), jnp.float32)
                return carry2
            return lax.fori_loop(0, D // L, _zcol, carry)
        lax.fori_loop(0, K, _zrow, 0)
        for b in range(rows_per_tile // K):
            pltpu.sync_copy(gbuf.at[2], acc.at[pl.ds(s * rows_per_tile + b * K, K)])
        plsc.subcore_barrier()

        def _chunk(i, carry):
            q = lax.rem(i, 5)
            p = lax.rem(i, 4)
            q2 = lax.rem(i + 2, 5)
            p2 = lax.rem(i + 2, 4)

            @pl.when(i >= 2)
            def _drain_old_scatter():
                _scatter_desc(lax.rem(i + 3, 5), p2).wait()

            @pl.when(i + 2 < chunks)
            def _start_gather_ahead():
                _idx_wait(i + 2, q2)
                _gather_start(i + 2, q2, p2)

            _gather_wait(i, q, p)
            _scatter_desc(q, p).start(add=True)

            @pl.when(i + 3 < chunks)
            def _prefetch_indices():
                _idx_start(i + 3, lax.rem(i + 3, 5))
            return carry
        lax.fori_loop(0, chunks, _chunk, 0)

        _scatter_desc((chunks - 2) % 5, (chunks - 2) % 4).wait()
        _scatter_desc((chunks - 1) % 5, (chunks - 1) % 4).wait()

        plsc.subcore_barrier()
        pltpu.sync_copy(
            acc.at[pl.ds(s * rows_per_tile, rows_per_tile)],
            out_hbm.at[c, pl.ds(s * rows_per_tile, rows_per_tile)],
        )

    return sc_scatter


def kernel(x, adj, weight, eps):
    n = x.shape[0]
    e = adj.shape[1]
    npad = ((n + NS * K - 1) // (NS * K)) * (NS * K)
    per_w = e // NW
    chunks = per_w // K
    assert per_w * NW == e and chunks * K == per_w

    wx = _matmul(x, weight)
    rc = adj.reshape(2, NW * chunks, K).transpose(1, 0, 2)
    partial = _make_sc_scatter(npad, chunks)(wx, rc)
    return _combine(partial, wx, eps.reshape(1, 1))


# speedup vs baseline: 1.0697x; 1.0697x over previous
import functools

import jax
import jax.numpy as jnp
from jax import lax
from jax.experimental import pallas as pl
from jax.experimental.pallas import tpu as pltpu
from jax.experimental.pallas import tpu_sc as plsc

D = 128
L = 16
NC = 2
NS = 16
NW = NC * NS
K = 80


def _mm_body(x_ref, w_ref, o_ref):
    o_ref[...] = jnp.dot(x_ref[...], w_ref[...], preferred_element_type=jnp.float32)


def _matmul(x, w):
    n = x.shape[0]
    bm = 2000
    return pl.pallas_call(
        _mm_body,
        grid=(n // bm,),
        in_specs=[
            pl.BlockSpec((bm, D), lambda i: (i, 0)),
            pl.BlockSpec((D, D), lambda i: (0, 0)),
        ],
        out_specs=pl.BlockSpec((bm, D), lambda i: (i, 0)),
        out_shape=jax.ShapeDtypeStruct((n, D), jnp.float32),
    )(x, w)


def _combine_body(eps_ref, p_ref, wx_ref, o_ref):
    agg = p_ref[0].astype(jnp.float32) + p_ref[1].astype(jnp.float32)
    o_ref[...] = agg + eps_ref[0, 0] * wx_ref[...]


def _combine(p, wx, eps):
    n = wx.shape[0]
    bm = 1000
    return pl.pallas_call(
        _combine_body,
        grid=(n // bm,),
        in_specs=[
            pl.BlockSpec(memory_space=pltpu.SMEM),
            pl.BlockSpec((2, bm, D), lambda i: (0, i, 0)),
            pl.BlockSpec((bm, D), lambda i: (i, 0)),
        ],
        out_specs=pl.BlockSpec((bm, D), lambda i: (i, 0)),
        out_shape=jax.ShapeDtypeStruct((n, D), jnp.float32),
    )(eps, p, wx)


def _make_sc_scatter(npad, chunks):
    rows_per_tile = npad // NS
    mesh = plsc.VectorSubcoreMesh(core_axis_name="c", subcore_axis_name="s")

    @functools.partial(
        pl.kernel,
        out_type=jax.ShapeDtypeStruct((NC, npad, D), jnp.float32),
        mesh=mesh,
        scratch_types=[
            pltpu.VMEM((5, 2, K), jnp.int32),
            pltpu.VMEM((4, K, D), jnp.float32),
            pltpu.VMEM_SHARED((npad, D), jnp.float32),
            pltpu.SemaphoreType.DMA((5,)),
            pltpu.SemaphoreType.DMA((4,)),
            pltpu.SemaphoreType.DMA((4,)),
        ],
    )
    def sc_scatter(wx_hbm, rc_hbm, out_hbm, ibuf, gbuf, acc, isem, gsem, ssem):
        c = lax.axis_index("c")
        s = lax.axis_index("s")
        w = s * NC + c

        def _idx_start(j, q):
            pltpu.async_copy(rc_hbm.at[w, j], ibuf.at[q], isem.at[q])

        def _idx_wait(j, q):
            pltpu.make_async_copy(rc_hbm.at[w, j], ibuf.at[q], isem.at[q]).wait()

        def _gather_start(j, q, p):
            pltpu.async_copy(wx_hbm.at[ibuf.at[q, 1]], gbuf.at[p], gsem.at[p])

        def _gather_wait(j, q, p):
            pltpu.make_async_copy(wx_hbm.at[ibuf.at[q, 1]], gbuf.at[p], gsem.at[p]).wait()

        def _scatter_desc(q, p):
            return pltpu.make_async_copy(gbuf.at[p], acc.at[ibuf.at[q, 0]], ssem.at[p])

        _idx_start(0, 0)
        _idx_start(1, 1)
        _idx_wait(0, 0)
        _gather_start(0, 0, 0)
        _idx_wait(1, 1)
        _gather_start(1, 1, 1)
        _idx_start(2, 2)

        def _zrow(i, carry):
            def _zcol(j, carry2):
                gbuf[2, i, pl.ds(j * L, L)] = jnp.zeros((L,), jnp.float32)
                return carry2
            return lax.fori_loop(0, D // L, _zcol, carry)
        lax.fori_loop(0, K, _zrow, 0)
        for b in range(rows_per_tile // K):
            pltpu.sync_copy(gbuf.at[2], acc.at[pl.ds(s * rows_per_tile + b * K, K)])
        plsc.subcore_barrier()

        def _chunk(i, carry):
            q = lax.rem(i, 5)
            p = lax.rem(i, 4)
            q2 = lax.rem(i + 2, 5)
            p2 = lax.rem(i + 2, 4)

            @pl.when(i >= 2)
            def _drain_old_scatter():
                _scatter_desc(lax.rem(i + 3, 5), p2).wait()

            @pl.when(i + 2 < chunks)
            def _start_gather_ahead():
                _idx_wait(i + 2, q2)
                _gather_start(i + 2, q2, p2)

            _gather_wait(i, q, p)
            _scatter_desc(q, p).start(add=True)

            @pl.when(i + 3 < chunks)
            def _prefetch_indices():
                _idx_start(i + 3, lax.rem(i + 3, 5))
            return carry
        lax.fori_loop(0, chunks, _chunk, 0)

        _scatter_desc((chunks - 2) % 5, (chunks - 2) % 4).wait()
        _scatter_desc((chunks - 1) % 5, (chunks - 1) % 4).wait()

        plsc.subcore_barrier()
        pltpu.sync_copy(
            acc.at[pl.ds(s * rows_per_tile, rows_per_tile)],
            out_hbm.at[c, pl.ds(s * rows_per_tile, rows_per_tile)],
        )

    return sc_scatter


def kernel(x, adj, weight, eps):
    n = x.shape[0]
    e = adj.shape[1]
    npad = ((n + NS * K - 1) // (NS * K)) * (NS * K)
    per_w = e // NW
    chunks = per_w // K
    assert per_w * NW == e and chunks * K == per_w

    wx = _matmul(x, weight)
    rc = adj.reshape(2, NW, chunks, K).transpose(1, 2, 0, 3)
    partial = _make_sc_scatter(npad, chunks)(wx, rc)
    return _combine(partial, wx, eps.reshape(1, 1))
